# Initial kernel scaffold; baseline (speedup 1.0000x reference)
import functools

import jax
import jax.numpy as jnp
from jax import lax
from jax.experimental import pallas as pl
from jax.experimental.pallas import tpu as pltpu
from jax.experimental.pallas import tpu_sc as plsc

N = 10000
E = 320000
IN = 128
HID = 128
HEADS = 4
NCLS = 16
NGRAPH = 64
HD = HEADS * HID
DR = 144

BN = 400
NBLK = N // BN

NTILES = 16
CH = E // NTILES
BK = 80
ZR = 125
NPT = N // NTILES


def _prep_body(x_ref, w_ref, attn_ref, fx_ref, elr_ref):
    xb = x_ref[...]
    wc = w_ref[...]
    for l in range(2):
        feat = jnp.dot(xb, wc[:, l * HD:(l + 1) * HD],
                       preferred_element_type=jnp.float32)
        featr = feat.reshape(BN, HEADS, HID)
        al = attn_ref[l * 8:l * 8 + 4, :]
        ar = attn_ref[l * 8 + 4:l * 8 + 8, :]
        el = jnp.sum(featr * al[None], -1)
        er = jnp.sum(featr * ar[None], -1)
        elr_ref[l * 8:l * 8 + 4, :] = el.T
        elr_ref[l * 8 + 4:l * 8 + 8, :] = er.T
        fx_ref[l, :, :, 0:HID] = featr.transpose(1, 0, 2)
        pad = jnp.concatenate(
            [jnp.ones((HEADS, BN, 1), jnp.float32),
             jnp.zeros((HEADS, BN, DR - HID - 1), jnp.float32)], -1)
        fx_ref[l, :, :, HID:DR] = pad


_prep_call = pl.pallas_call(
    _prep_body,
    grid=(NBLK,),
    in_specs=[
        pl.BlockSpec((BN, IN), lambda i: (i, 0)),
        pl.BlockSpec((IN, 2 * HD), lambda i: (0, 0)),
        pl.BlockSpec((16, HID), lambda i: (0, 0)),
    ],
    out_specs=[
        pl.BlockSpec((2, HEADS, BN, DR), lambda i: (0, 0, i, 0)),
        pl.BlockSpec((16, BN), lambda i: (0, i)),
    ],
    out_shape=[
        jax.ShapeDtypeStruct((2, HEADS, N, DR), jnp.float32),
        jax.ShapeDtypeStruct((16, N), jnp.float32),
    ],
)


def _sc_body(featx, elr, ei1, ei2, out,
             src_v, dst_v, el_v, er_v, idx_v, dstb_v, ex_v, gbuf, zbuf, acc, sem):
    cid = lax.axis_index("c")
    sid = lax.axis_index("s")
    ebase = sid * CH
    rbase = sid * NPT

    z16 = jnp.zeros((16,), jnp.float32)
    for k in range(ZR):
        for j in range(DR // 16):
            zbuf[k, pl.ds(j * 16, 16)] = z16

    for p in range(4):
        layer = p // 2
        hl = p % 2
        head = 2 * cid + hl
        foff = (layer * 4 + head) * N

        ei = ei1 if layer == 0 else ei2
        pltpu.sync_copy(ei.at[0, pl.ds(ebase, CH)], src_v)
        pltpu.sync_copy(ei.at[1, pl.ds(ebase, CH)], dst_v)
        pltpu.sync_copy(elr.at[pl.ds((layer * 8 + head) * N, N)], el_v)
        pltpu.sync_copy(elr.at[pl.ds((layer * 8 + 4 + head) * N, N)], er_v)

        for c in range(NPT // ZR):
            pltpu.sync_copy(zbuf, acc.at[pl.ds(rbase + c * ZR, ZR)])
        plsc.subcore_barrier()

        def blk_body(b, carry):
            eoff = b * BK
            for i in range(BK // 16):
                s16 = src_v[pl.ds(eoff + i * 16, 16)]
                d16 = dst_v[pl.ds(eoff + i * 16, 16)]
                el_s = plsc.load_gather(el_v, [s16])
                er_d = plsc.load_gather(er_v, [d16])
                e = el_s + er_d
                e = jnp.where(e > 0, e, 0.2 * e)
                ex_v[pl.ds(i * 16, 16)] = jnp.exp(e)
                idx_v[pl.ds(i * 16, 16)] = s16 + foff
                dstb_v[pl.ds(i * 16, 16)] = d16
            pltpu.async_copy(featx.at[idx_v], gbuf, sem).wait()

            def row_body(k, carry2):
                exk = plsc.load_gather(ex_v, [jnp.full((16,), 0, jnp.int32) + k])
                for j in range(DR // 16):
                    gbuf[k, pl.ds(j * 16, 16)] = gbuf[k, pl.ds(j * 16, 16)] * exk
                return carry2

            lax.fori_loop(0, BK, row_body, 0)
            pltpu.sync_copy(gbuf, acc.at[dstb_v], add=True)
            return carry

        lax.fori_loop(0, CH // BK, blk_body, 0)
        plsc.subcore_barrier()

        for c in range(NPT // ZR):
            roff = foff + rbase + c * ZR
            pltpu.sync_copy(acc.at[pl.ds(rbase + c * ZR, ZR)],
                            out.at[pl.ds(roff, ZR)])
        plsc.subcore_barrier()


_sc_call = functools.partial(
    pl.kernel,
    mesh=plsc.VectorSubcoreMesh(core_axis_name="c", subcore_axis_name="s"),
    out_type=jax.ShapeDtypeStruct((8 * N, DR), jnp.float32),
    scratch_types=[
        pltpu.VMEM((CH,), jnp.int32),
        pltpu.VMEM((CH,), jnp.int32),
        pltpu.VMEM((N,), jnp.float32),
        pltpu.VMEM((N,), jnp.float32),
        pltpu.VMEM((BK,), jnp.int32),
        pltpu.VMEM((BK,), jnp.int32),
        pltpu.VMEM((BK,), jnp.float32),
        pltpu.VMEM((BK, DR), jnp.float32),
        pltpu.VMEM((ZR, DR), jnp.float32),
        pltpu.VMEM_SHARED((N, DR), jnp.float32),
        pltpu.SemaphoreType.DMA,
    ],
)(_sc_body)


def _final_body(acc1_ref, acc2_ref, gid_ref, saw1_ref, sab1_ref, saw2_ref,
                clsw_ref, clsb_ref, out_ref, p1_acc, p2_acc, cnt_acc, w_acc):
    i = pl.program_id(0)

    @pl.when(i == 0)
    def _():
        p1_acc[...] = jnp.zeros_like(p1_acc)
        p2_acc[...] = jnp.zeros_like(p2_acc)
        cnt_acc[...] = jnp.zeros_like(cnt_acc)
        w_acc[0] = 0.0
        w_acc[1] = 0.0

    gidb = gid_ref[0, 0, :]
    onehot = (gidb[:, None] == lax.broadcasted_iota(jnp.float32, (1, NGRAPH), 1)
              ).astype(jnp.float32)
    ones_col = jnp.ones((BN, 128), jnp.float32)
    cnt_acc[...] += jnp.einsum("ng,nd->gd", onehot, ones_col,
                               preferred_element_type=jnp.float32)

    saw1 = saw1_ref[...]
    sab1 = sab1_ref[...]
    saw2 = saw2_ref[...]
    for l, (acc_ref, p_acc) in enumerate(((acc1_ref, p1_acc), (acc2_ref, p2_acc))):
        blk = acc_ref[...]
        denom = blk[:, :, HID:HID + 1] + 1e-9
        h = blk[:, :, 0:HID] / denom
        h = jnp.where(h > 0, h, jnp.exp(jnp.minimum(h, 0.0)) - 1.0)
        h = h.transpose(1, 0, 2).reshape(BN, HD)
        t = jnp.tanh(jnp.dot(h, saw1, preferred_element_type=jnp.float32)
                     + sab1[0:1, :])
        wcol = jnp.dot(t, saw2, preferred_element_type=jnp.float32)
        w_acc[l] += jnp.sum(wcol)
        p_acc[...] += jnp.einsum("ng,nd->gd", onehot, h,
                                 preferred_element_type=jnp.float32)

    @pl.when(i == NBLK - 1)
    def _():
        w0 = w_acc[0] / N
        w1 = w_acc[1] / N
        m = jnp.maximum(w0, w1)
        e0 = jnp.exp(w0 - m)
        e1 = jnp.exp(w1 - m)
        b0 = e0 / (e0 + e1)
        b1 = e1 / (e0 + e1)
        cnt = jnp.maximum(cnt_acc[:, 0:1], 1.0)
        pooled = (b0 * p1_acc[...] + b1 * p2_acc[...]) / cnt
        out_ref[...] = (jnp.dot(pooled, clsw_ref[...],
                                preferred_element_type=jnp.float32)
                        + clsb_ref[0:1, :])


_final_call = pl.pallas_call(
    _final_body,
    grid=(NBLK,),
    in_specs=[
        pl.BlockSpec((HEADS, BN, DR), lambda i: (0, i, 0)),
        pl.BlockSpec((HEADS, BN, DR), lambda i: (0, i, 0)),
        pl.BlockSpec((1, 1, BN), lambda i: (i, 0, 0)),
        pl.BlockSpec((HD, 128), lambda i: (0, 0)),
        pl.BlockSpec((1, 128), lambda i: (0, 0)),
        pl.BlockSpec((128, 1), lambda i: (0, 0)),
        pl.BlockSpec((HD, NCLS), lambda i: (0, 0)),
        pl.BlockSpec((1, NCLS), lambda i: (0, 0)),
    ],
    out_specs=pl.BlockSpec((NGRAPH, NCLS), lambda i: (0, 0)),
    out_shape=jax.ShapeDtypeStruct((NGRAPH, NCLS), jnp.float32),
    scratch_shapes=[
        pltpu.VMEM((NGRAPH, HD), jnp.float32),
        pltpu.VMEM((NGRAPH, HD), jnp.float32),
        pltpu.VMEM((NGRAPH, 128), jnp.float32),
        pltpu.SMEM((2,), jnp.float32),
    ],
    compiler_params=pltpu.CompilerParams(
        dimension_semantics=("arbitrary",)),
)


def kernel(x, edge_index1, edge_index2, graph_ids, W1, al1, ar1, W2, al2, ar2,
           sa_W1, sa_b1, sa_w2, cls_W, cls_b):
    wc = jnp.concatenate([W1, W2], axis=1)
    attn = jnp.concatenate([al1, ar1, al2, ar2], axis=0)
    featx, elr = _prep_call(x, wc, attn)
    featx_flat = featx.reshape(2 * HEADS * N, DR)
    elr_flat = elr.reshape(16 * N)
    ei1 = edge_index1.astype(jnp.int32)
    ei2 = edge_index2.astype(jnp.int32)
    accs = _sc_call(featx_flat, elr_flat, ei1, ei2)
    accs = accs.reshape(2, HEADS, N, DR)
    gidf = graph_ids.astype(jnp.float32).reshape(NBLK, 1, BN)
    out = _final_call(accs[0], accs[1], gidf, sa_W1,
                      sa_b1.reshape(1, 128), sa_w2, cls_W, cls_b.reshape(1, NCLS))
    return out


# speedup vs baseline: 18.4409x; 18.4409x over previous
import functools

import jax
import jax.numpy as jnp
from jax import lax
from jax.experimental import pallas as pl
from jax.experimental.pallas import tpu as pltpu
from jax.experimental.pallas import tpu_sc as plsc

N = 10000
E = 320000
IN = 128
HID = 128
HEADS = 4
NCLS = 16
NGRAPH = 64
HD = HEADS * HID
DR = 144

BN = 400
NBLK = N // BN

NTILES = 16
CH = E // NTILES
SUB = 2000
BK = 80
N_PAD = 10240
NPT = N_PAD // NTILES
ZR = 80


def _prep_body(x_ref, w_ref, attn_ref, fx_ref, elr_ref):
    xb = x_ref[...]
    wc = w_ref[...]
    for l in range(2):
        feat = jnp.dot(xb, wc[:, l * HD:(l + 1) * HD],
                       preferred_element_type=jnp.float32)
        featr = feat.reshape(BN, HEADS, HID)
        al = attn_ref[l * 8:l * 8 + 4, :]
        ar = attn_ref[l * 8 + 4:l * 8 + 8, :]
        el = jnp.sum(featr * al[None], -1)
        er = jnp.sum(featr * ar[None], -1)
        elr_ref[0, l * 8:l * 8 + 4, :] = el.T
        elr_ref[0, l * 8 + 4:l * 8 + 8, :] = er.T
        fx_ref[l, :, :, 0:HID] = featr.transpose(1, 0, 2)
        pad = jnp.concatenate(
            [jnp.ones((HEADS, BN, 1), jnp.float32),
             jnp.zeros((HEADS, BN, DR - HID - 1), jnp.float32)], -1)
        fx_ref[l, :, :, HID:DR] = pad


_prep_call = pl.pallas_call(
    _prep_body,
    grid=(NBLK,),
    in_specs=[
        pl.BlockSpec((BN, IN), lambda i: (i, 0)),
        pl.BlockSpec((IN, 2 * HD), lambda i: (0, 0)),
        pl.BlockSpec((16, HID), lambda i: (0, 0)),
    ],
    out_specs=[
        pl.BlockSpec((2, HEADS, BN, DR), lambda i: (0, 0, i, 0)),
        pl.BlockSpec((1, 16, BN), lambda i: (i, 0, 0)),
    ],
    out_shape=[
        jax.ShapeDtypeStruct((2, HEADS, N, DR), jnp.float32),
        jax.ShapeDtypeStruct((NBLK, 16, BN), jnp.float32),
    ],
)


def _sc_body(featx, elr, ei1, ei2, out,
             src_v, dst_v, el_v, er_v, idx_v, dstb_v, ex_v, gbuf, acc, sem):
    cid = lax.axis_index("c")
    sid = lax.axis_index("s")
    ebase = sid * CH
    rbase = sid * NPT
    z16 = jnp.zeros((16,), jnp.float32)

    for p in range(4):
        layer = p // 2
        hl = p % 2
        head = 2 * cid + hl
        foff = (layer * 4 + head) * N
        ooff = (layer * 4 + head) * N_PAD

        ei = ei1 if layer == 0 else ei2
        pltpu.sync_copy(elr.at[pl.ds((layer * 8 + head) * N, N)], el_v)
        pltpu.sync_copy(elr.at[pl.ds((layer * 8 + 4 + head) * N, N)], er_v)

        for k in range(ZR):
            for j in range(DR // 16):
                gbuf[k, pl.ds(j * 16, 16)] = z16
        for c in range(NPT // ZR):
            pltpu.sync_copy(gbuf, acc.at[pl.ds(rbase + c * ZR, ZR)])
        plsc.subcore_barrier()

        def sb_body(sb, carry):
            soff = ebase + sb * SUB
            pltpu.sync_copy(ei.at[pl.ds(soff, SUB)], src_v)
            pltpu.sync_copy(ei.at[pl.ds(E + soff, SUB)], dst_v)

            def blk_body(b, carry1):
                eoff = b * BK
                for i in range(BK // 16):
                    s16 = src_v[pl.ds(eoff + i * 16, 16)]
                    d16 = dst_v[pl.ds(eoff + i * 16, 16)]
                    el_s = plsc.load_gather(el_v, [s16])
                    er_d = plsc.load_gather(er_v, [d16])
                    e = el_s + er_d
                    e = jnp.where(e > 0, e, 0.2 * e)
                    ex_v[pl.ds(i * 16, 16)] = jnp.exp(e)
                    idx_v[pl.ds(i * 16, 16)] = s16 + foff
                    dstb_v[pl.ds(i * 16, 16)] = d16
                pltpu.async_copy(featx.at[idx_v], gbuf, sem).wait()

                def row_body(k, carry2):
                    exk = plsc.load_gather(ex_v,
                                           [jnp.full((16,), 0, jnp.int32) + k])
                    for j in range(DR // 16):
                        gbuf[k, pl.ds(j * 16, 16)] = (
                            gbuf[k, pl.ds(j * 16, 16)] * exk)
                    return carry2

                lax.fori_loop(0, BK, row_body, 0)
                pltpu.sync_copy(gbuf, acc.at[dstb_v], add=True)
                return carry1

            lax.fori_loop(0, SUB // BK, blk_body, 0)
            return carry

        lax.fori_loop(0, CH // SUB, sb_body, 0)
        plsc.subcore_barrier()

        for c in range(NPT // ZR):
            roff = ooff + rbase + c * ZR
            pltpu.sync_copy(acc.at[pl.ds(rbase + c * ZR, ZR)],
                            out.at[pl.ds(roff, ZR)])
        plsc.subcore_barrier()


_sc_call = functools.partial(
    pl.kernel,
    mesh=plsc.VectorSubcoreMesh(core_axis_name="c", subcore_axis_name="s"),
    out_type=jax.ShapeDtypeStruct((8 * N_PAD, DR), jnp.float32),
    scratch_types=[
        pltpu.VMEM((SUB,), jnp.int32),
        pltpu.VMEM((SUB,), jnp.int32),
        pltpu.VMEM((N,), jnp.float32),
        pltpu.VMEM((N,), jnp.float32),
        pltpu.VMEM((BK,), jnp.int32),
        pltpu.VMEM((BK,), jnp.int32),
        pltpu.VMEM((BK,), jnp.float32),
        pltpu.VMEM((BK, DR), jnp.float32),
        pltpu.VMEM_SHARED((N_PAD, DR), jnp.float32),
        pltpu.SemaphoreType.DMA,
    ],
    compiler_params=pltpu.CompilerParams(needs_layout_passes=False,
                                         use_tc_tiling_on_sc=False),
)(_sc_body)


def _final_body(acc1_ref, acc2_ref, gid_ref, saw1_ref, sab1_ref, saw2_ref,
                clsw_ref, clsb_ref, out_ref, p1_acc, p2_acc, cnt_acc, w_acc):
    i = pl.program_id(0)

    @pl.when(i == 0)
    def _():
        p1_acc[...] = jnp.zeros_like(p1_acc)
        p2_acc[...] = jnp.zeros_like(p2_acc)
        cnt_acc[...] = jnp.zeros_like(cnt_acc)
        w_acc[0] = 0.0
        w_acc[1] = 0.0

    gidb = gid_ref[0, 0, :]
    iotag = lax.broadcasted_iota(jnp.int32, (1, NGRAPH), 1).astype(jnp.float32)
    onehot = (gidb[:, None] == iotag).astype(jnp.float32)
    ones_col = jnp.ones((BN, 128), jnp.float32)
    cnt_acc[...] += jnp.einsum("ng,nd->gd", onehot, ones_col,
                               preferred_element_type=jnp.float32)

    saw1 = saw1_ref[...]
    sab1 = sab1_ref[...]
    saw2 = saw2_ref[...]
    for l, (acc_ref, p_acc) in enumerate(((acc1_ref, p1_acc), (acc2_ref, p2_acc))):
        blk = acc_ref[...]
        denom = blk[:, :, HID:HID + 1] + 1e-9
        h = blk[:, :, 0:HID] / denom
        h = jnp.where(h > 0, h, jnp.exp(jnp.minimum(h, 0.0)) - 1.0)
        h = h.transpose(1, 0, 2).reshape(BN, HD)
        t = jnp.tanh(jnp.dot(h, saw1, preferred_element_type=jnp.float32)
                     + sab1[0:1, :])
        wcol = jnp.dot(t, saw2, preferred_element_type=jnp.float32)
        w_acc[l] += jnp.sum(wcol)
        p_acc[...] += jnp.einsum("ng,nd->gd", onehot, h,
                                 preferred_element_type=jnp.float32)

    @pl.when(i == NBLK - 1)
    def _():
        w0 = w_acc[0] / N
        w1 = w_acc[1] / N
        m = jnp.maximum(w0, w1)
        e0 = jnp.exp(w0 - m)
        e1 = jnp.exp(w1 - m)
        b0 = e0 / (e0 + e1)
        b1 = e1 / (e0 + e1)
        cnt = jnp.maximum(cnt_acc[:, 0:1], 1.0)
        pooled = (b0 * p1_acc[...] + b1 * p2_acc[...]) / cnt
        out_ref[...] = (jnp.dot(pooled, clsw_ref[...],
                                preferred_element_type=jnp.float32)
                        + clsb_ref[0:1, :])


_final_call = pl.pallas_call(
    _final_body,
    grid=(NBLK,),
    in_specs=[
        pl.BlockSpec((HEADS, BN, DR), lambda i: (0, i, 0)),
        pl.BlockSpec((HEADS, BN, DR), lambda i: (0, i, 0)),
        pl.BlockSpec((1, 1, BN), lambda i: (i, 0, 0)),
        pl.BlockSpec((HD, 128), lambda i: (0, 0)),
        pl.BlockSpec((1, 128), lambda i: (0, 0)),
        pl.BlockSpec((128, 1), lambda i: (0, 0)),
        pl.BlockSpec((HD, NCLS), lambda i: (0, 0)),
        pl.BlockSpec((1, NCLS), lambda i: (0, 0)),
    ],
    out_specs=pl.BlockSpec((NGRAPH, NCLS), lambda i: (0, 0)),
    out_shape=jax.ShapeDtypeStruct((NGRAPH, NCLS), jnp.float32),
    scratch_shapes=[
        pltpu.VMEM((NGRAPH, HD), jnp.float32),
        pltpu.VMEM((NGRAPH, HD), jnp.float32),
        pltpu.VMEM((NGRAPH, 128), jnp.float32),
        pltpu.SMEM((2,), jnp.float32),
    ],
    compiler_params=pltpu.CompilerParams(
        dimension_semantics=("arbitrary",)),
)


def kernel(x, edge_index1, edge_index2, graph_ids, W1, al1, ar1, W2, al2, ar2,
           sa_W1, sa_b1, sa_w2, cls_W, cls_b):
    wc = jnp.concatenate([W1, W2], axis=1)
    attn = jnp.concatenate([al1, ar1, al2, ar2], axis=0)
    featx, elr = _prep_call(x, wc, attn)
    featx_flat = featx.reshape(2 * HEADS * N, DR)
    elr_flat = elr.transpose(1, 0, 2).reshape(16 * N)
    ei1 = edge_index1.astype(jnp.int32).reshape(2 * E)
    ei2 = edge_index2.astype(jnp.int32).reshape(2 * E)
    accs = _sc_call(featx_flat, elr_flat, ei1, ei2)
    accs = accs.reshape(2, HEADS, N_PAD, DR)[:, :, :N, :]
    gidf = graph_ids.astype(jnp.float32).reshape(NBLK, 1, BN)
    out = _final_call(accs[0], accs[1], gidf, sa_W1,
                      sa_b1.reshape(1, 128), sa_w2, cls_W, cls_b.reshape(1, NCLS))
    return out
